# Initial kernel scaffold; baseline (speedup 1.0000x reference)
import jax
import jax.numpy as jnp
from jax.experimental import pallas as pl


def kernel(x, W, W_out):
    raise NotImplementedError("write your pallas kernel here")


# speedup vs baseline: 19.9202x; 19.9202x over previous
import functools

import jax
import jax.numpy as jnp
from jax import lax
from jax.experimental import pallas as pl
from jax.experimental.pallas import tpu as pltpu

_N_VPN = 64
_N_KC = 32768
_K_TOP = 1638
_ROWS_PER_BLOCK = 32
_HI_BITS = 0x3F800000
_N_ITERS = 30


def _mb_body(k_top, hi_bits, n_iters, x_ref, wt_ref, wout_ref, mask_ref,
             mbon_ref, bits_ref):
    logits = jnp.dot(x_ref[...], wt_ref[...],
                     preferred_element_type=jnp.float32)
    bits_ref[...] = lax.bitcast_convert_type(logits, jnp.int32)
    rows = x_ref.shape[0]
    lo0 = jnp.full((rows, 1), -1, jnp.int32)
    hi0 = jnp.full((rows, 1), hi_bits, jnp.int32)

    def step(_, carry):
        lo, hi = carry
        mid = lax.shift_right_arithmetic(lo + hi, 1)
        gt = (bits_ref[...] > mid).astype(jnp.int32)
        cnt = jnp.sum(gt, axis=1, keepdims=True)
        pred = cnt < k_top
        return jnp.where(pred, lo, mid), jnp.where(pred, mid, hi)

    _, thr = lax.fori_loop(0, n_iters, step, (lo0, hi0))

    bits = bits_ref[...]
    gt = bits > thr
    eq = bits == thr
    eq_i = eq.astype(jnp.int32)
    g = jnp.sum(gt.astype(jnp.int32), axis=1, keepdims=True)
    r = k_top - g
    n_kc = bits.shape[1]
    col = lax.broadcasted_iota(jnp.int32, bits.shape, 1)
    clo0 = jnp.full((rows, 1), -1, jnp.int32)
    chi0 = jnp.full((rows, 1), n_kc - 1, jnp.int32)

    def cstep(_, carry):
        lo, hi = carry
        mid = lax.shift_right_arithmetic(lo + hi, 1)
        cnt = jnp.sum(eq_i * (col <= mid).astype(jnp.int32), axis=1,
                      keepdims=True)
        pred = cnt >= r
        return jnp.where(pred, lo, mid), jnp.where(pred, mid, hi)

    _, cthr = lax.fori_loop(0, 15, cstep, (clo0, chi0))
    maskf = (gt | (eq & (col <= cthr))).astype(jnp.float32)
    mask_ref[...] = maskf
    mbon_ref[...] = jnp.sum(maskf * wout_ref[...], axis=1, keepdims=True)


def _build(n_vpn, n_kc, k_top, rows_per_block, batch, hi_bits, n_iters,
           interpret=False):
    grid = batch // rows_per_block
    return pl.pallas_call(
        functools.partial(_mb_body, k_top, hi_bits, n_iters),
        grid=(grid,),
        in_specs=[
            pl.BlockSpec((rows_per_block, n_vpn), lambda i: (i, 0)),
            pl.BlockSpec((n_vpn, n_kc), lambda i: (0, 0)),
            pl.BlockSpec((1, n_kc), lambda i: (0, 0)),
        ],
        out_specs=[
            pl.BlockSpec((rows_per_block, n_kc), lambda i: (i, 0)),
            pl.BlockSpec((rows_per_block, 1), lambda i: (i, 0)),
        ],
        out_shape=[
            jax.ShapeDtypeStruct((batch, n_kc), jnp.float32),
            jax.ShapeDtypeStruct((batch, 1), jnp.float32),
        ],
        scratch_shapes=[pltpu.VMEM((rows_per_block, n_kc), jnp.int32)],
        interpret=interpret,
    )


def kernel(x, W, W_out):
    batch = x.shape[0]
    wt = W.T
    mask, mbon = _build(_N_VPN, _N_KC, _K_TOP, _ROWS_PER_BLOCK, batch,
                        _HI_BITS, _N_ITERS)(x, wt, W_out)
    return (mask, mbon)
